# baseline (speedup 1.0000x reference)
import functools

import jax
import jax.numpy as jnp
from jax import lax
from jax.experimental import pallas as pl
from jax.experimental.pallas import tpu as pltpu
from jax.experimental.pallas import tpu_sc as plsc

_NC = 2
_NS = 16
_L = 16
_NW = _NC * _NS
_B = 16384
_F = 64
_BPW = _B // _NW
_CH = 128
_NCH = _BPW // _CH
_G = _BPW // _L


def _body(p_ref, poll_ref, pe_hbm, pb_hbm, qe_hbm, qb_hbm, out_hbm,
          idx_p, idx_q, pe_v, qe_v, pb_v, qb_v, out_v, sem):
    c = lax.axis_index("c")
    s = lax.axis_index("s")
    wid = s * _NC + c
    row0 = wid * _NCH
    base = wid * _BPW

    pltpu.sync_copy(p_ref.at[pl.ds(row0, _NCH)], idx_p)
    pltpu.sync_copy(poll_ref.at[pl.ds(row0, _NCH)], idx_q)

    copies = []
    for j in range(_NCH):
        dst = pl.ds(j * _CH, _CH)
        copies.append(pltpu.async_copy(pe_hbm.at[idx_p.at[j]], pe_v.at[dst], sem))
        copies.append(pltpu.async_copy(qe_hbm.at[idx_q.at[j]], qe_v.at[dst], sem))
        copies.append(pltpu.async_copy(pb_hbm.at[idx_p.at[j]], pb_v.at[dst], sem))
        copies.append(pltpu.async_copy(qb_hbm.at[idx_q.at[j]], qb_v.at[dst], sem))
    for cp in copies:
        cp.wait()

    iota = lax.iota(jnp.int32, _L)
    zero16 = jnp.zeros((_L,), jnp.int32)

    def group(g, carry):
        dots = jnp.zeros((_L,), jnp.float32)
        for r in range(_L):
            row = g * _L + r
            s_acc = None
            for k in range(_F // _L):
                a = pe_v[row, pl.ds(k * _L, _L)]
                b = qe_v[row, pl.ds(k * _L, _L)]
                s_acc = a * b if s_acc is None else s_acc + a * b
            dot = jnp.sum(s_acc)
            dots = jnp.where(iota == r, dot, dots)
        rows = g * _L + iota
        pb = plsc.load_gather(pb_v, [rows, zero16])
        qb = plsc.load_gather(qb_v, [rows, zero16])
        x = dots + pb + qb
        out_v[pl.ds(g * _L, _L)] = 1.0 / (1.0 + jnp.exp(-x))
        return carry

    lax.fori_loop(0, _G, group, 0)

    pltpu.sync_copy(out_v, out_hbm.at[pl.ds(base, _BPW)])


@jax.jit
def kernel(p, poll, p_embed, p_bias, poll_embed, poll_bias):
    p2 = p.astype(jnp.int32).reshape(_NW * _NCH, _CH)
    q2 = poll.astype(jnp.int32).reshape(_NW * _NCH, _CH)
    mesh = plsc.VectorSubcoreMesh(core_axis_name="c", subcore_axis_name="s")
    run = pl.kernel(
        _body,
        mesh=mesh,
        compiler_params=pltpu.CompilerParams(
            needs_layout_passes=False, use_tc_tiling_on_sc=False),
        out_type=jax.ShapeDtypeStruct((_B,), jnp.float32),
        scratch_types=[
            pltpu.VMEM((_NCH, _CH), jnp.int32),
            pltpu.VMEM((_NCH, _CH), jnp.int32),
            pltpu.VMEM((_BPW, _F), jnp.float32),
            pltpu.VMEM((_BPW, _F), jnp.float32),
            pltpu.VMEM((_BPW, 1), jnp.float32),
            pltpu.VMEM((_BPW, 1), jnp.float32),
            pltpu.VMEM((_BPW,), jnp.float32),
            pltpu.SemaphoreType.DMA,
        ],
    )
    return run(p2, q2, p_embed, p_bias, poll_embed, poll_bias)


# speedup vs baseline: 1.2894x; 1.2894x over previous
import functools

import jax
import jax.numpy as jnp
from jax import lax
from jax.experimental import pallas as pl
from jax.experimental.pallas import tpu as pltpu
from jax.experimental.pallas import tpu_sc as plsc

_NC = 2
_NS = 16
_L = 16
_NW = _NC * _NS
_B = 16384
_F = 64
_N = 100000
_TC = 128
_NCOL = (_N + _TC - 1) // _TC
_CPW = 25
_LCAP = 720
_SCAP = 64
_BPW = _B // _NW
_CH = 128
_NCH = _BPW // _CH
_HB = _BPW // 2
_G = _HB // _L
_BD = _B + _TC
_SENT = 0x7FFF0000
_DUMMY = _B


def _k1_body(pT, qT, pidx_hbm, qidx_hbm, out_p, out_q,
             idx_v, lidx_p, lpos_p, lidx_q, lpos_q,
             slab_p, slab_q, ebuf_p, ebuf_q,
             csub_p, psub_p, csub_q, psub_q, sem_in, sem_out):
    c = lax.axis_index("c")
    s = lax.axis_index("s")
    wid = s * _NC + c
    lo = wid * _CPW * _TC
    hi = lo + _CPW * _TC
    iota = lax.iota(jnp.int32, _L)

    def fill(t, carry):
        off = pl.ds(t * _L, _L)
        sv = jnp.full((_L,), _SENT, jnp.int32)
        lidx_p[off] = sv
        lidx_q[off] = sv
        lpos_p[off] = sv
        lpos_q[off] = sv
        return carry
    lax.fori_loop(0, _LCAP // _L, fill, 0)

    pltpu.sync_copy(pidx_hbm, idx_v.at[0])
    pltpu.sync_copy(qidx_hbm, idx_v.at[1])

    def scan(t, carry):
        offp, offq = carry
        sl = pl.ds(t * _L, _L)
        pos = t * _L + iota
        op = jnp.minimum(offp, _LCAP - _L)
        oq = jnp.minimum(offq, _LCAP - _L)
        vp = idx_v[0, sl]
        mp = (vp >= lo) & (vp < hi)
        plsc.store_compressed(lidx_p.at[pl.ds(op, _L)], vp, mask=mp)
        plsc.store_compressed(lpos_p.at[pl.ds(op, _L)], pos, mask=mp)
        np_ = plsc.all_reduce_population_count(mp)[0]
        vq = idx_v[1, sl]
        mq = (vq >= lo) & (vq < hi)
        plsc.store_compressed(lidx_q.at[pl.ds(oq, _L)], vq, mask=mq)
        plsc.store_compressed(lpos_q.at[pl.ds(oq, _L)], pos, mask=mq)
        nq = plsc.all_reduce_population_count(mq)[0]
        return offp + np_, offq + nq

    lax.fori_loop(0, _B // _L, scan, (jnp.int32(0), jnp.int32(0)))

    def do_slab(cid, table, lidx, lpos, slab, ebuf, csub, psub, out_hbm):
        col0 = pl.multiple_of(cid * _TC, _TC)
        pltpu.async_copy(
            table.at[pl.ds(0, _F), pl.ds(col0, _TC)], slab, sem_in).wait()
        dums = jnp.full((_L,), _DUMMY, jnp.int32) + wid
        zeros = jnp.zeros((_L,), jnp.int32)
        for t in range(_SCAP // _L):
            psub[pl.ds(t * _L, _L)] = dums
            csub[pl.ds(t * _L, _L)] = zeros
        def filt(t, off2):
            sl = pl.ds(t * _L, _L)
            v = lidx[sl]
            po = lpos[sl]
            m = (v >= col0) & (v < col0 + _TC)
            o2 = jnp.minimum(off2, _SCAP - _L)
            plsc.store_compressed(csub.at[pl.ds(o2, _L)], v - col0, mask=m)
            plsc.store_compressed(psub.at[pl.ds(o2, _L)], po, mask=m)
            return off2 + plsc.all_reduce_population_count(m)[0]
        n2 = lax.fori_loop(0, _LCAP // _L, filt, jnp.int32(0))
        for t in range(_SCAP // _L):
            @pl.when(t * _L < n2)
            def _():
                cols = csub[pl.ds(t * _L, _L)]
                for r in range(_L):
                    colr = jnp.full((_L,), cols[r], jnp.int32)
                    for k in range(_F // _L):
                        vals = plsc.load_gather(slab, [k * _L + iota, colr])
                        ebuf[t * _L + r, pl.ds(k * _L, _L)] = vals
        pltpu.async_copy(ebuf, out_hbm.at[psub], sem_out).wait()

    def slab_loop(c_loc, carry):
        cid = jnp.minimum(wid * _CPW + c_loc, _NCOL - 1)
        do_slab(cid, pT, lidx_p, lpos_p, slab_p, ebuf_p, csub_p, psub_p, out_p)
        do_slab(cid, qT, lidx_q, lpos_q, slab_q, ebuf_q, csub_q, psub_q, out_q)
        return carry

    lax.fori_loop(0, _CPW, slab_loop, 0)


def _k2_body(rows_p_hbm, rows_q_hbm, p2_hbm, q2_hbm, pb_hbm, qb_hbm, out_hbm,
             idx_p, idx_q, pe_v, qe_v, pb_v, qb_v, out_v, sem):
    c = lax.axis_index("c")
    s = lax.axis_index("s")
    wid = s * _NC + c
    base = wid * _BPW

    pltpu.sync_copy(p2_hbm.at[wid], idx_p)
    pltpu.sync_copy(q2_hbm.at[wid], idx_q)

    bias_copies = []
    for j in range(_NCH):
        dst = pl.ds(j * _CH, _CH)
        bias_copies.append(
            pltpu.async_copy(pb_hbm.at[idx_p.at[j]], pb_v.at[dst], sem))
        bias_copies.append(
            pltpu.async_copy(qb_hbm.at[idx_q.at[j]], qb_v.at[dst], sem))

    iota = lax.iota(jnp.int32, _L)

    for h in range(2):
        rbase = base + h * _HB
        cp_p = pltpu.async_copy(
            rows_p_hbm.at[pl.ds(rbase, _HB)], pe_v, sem)
        cp_q = pltpu.async_copy(
            rows_q_hbm.at[pl.ds(rbase, _HB)], qe_v, sem)
        cp_p.wait()
        cp_q.wait()
        if h == 0:
            for cp in bias_copies:
                cp.wait()

        def group(g, carry):
            dots = jnp.zeros((_L,), jnp.float32)
            for r in range(_L):
                rloc = g * _L + r
                s_acc = None
                for k in range(_F // _L):
                    a = pe_v[rloc, pl.ds(k * _L, _L)]
                    b = qe_v[rloc, pl.ds(k * _L, _L)]
                    s_acc = a * b if s_acc is None else s_acc + a * b
                dot = jnp.sum(s_acc)
                dots = jnp.where(iota == r, dot, dots)
            off = pl.ds(h * _HB + g * _L, _L)
            x = dots + pb_v[off] + qb_v[off]
            out_v[off] = 1.0 / (1.0 + jnp.exp(-x))
            return carry

        lax.fori_loop(0, _G, group, 0)

    pltpu.sync_copy(out_v, out_hbm.at[pl.ds(base, _BPW)])


@jax.jit
def kernel(p, poll, p_embed, p_bias, poll_embed, poll_bias):
    pi = p.astype(jnp.int32)
    qi = poll.astype(jnp.int32)
    p2 = pi.reshape(_NW, _NCH, _CH)
    q2 = qi.reshape(_NW, _NCH, _CH)
    pT = p_embed.T
    qT = poll_embed.T
    mesh = plsc.VectorSubcoreMesh(core_axis_name="c", subcore_axis_name="s")
    params = pltpu.CompilerParams(
        needs_layout_passes=False, use_tc_tiling_on_sc=True)

    k1 = pl.kernel(
        _k1_body,
        mesh=mesh,
        compiler_params=params,
        out_type=(
            jax.ShapeDtypeStruct((_BD, _TC), jnp.float32),
            jax.ShapeDtypeStruct((_BD, _TC), jnp.float32),
        ),
        scratch_types=[
            pltpu.VMEM((2, _B), jnp.int32),
            pltpu.VMEM((_LCAP,), jnp.int32),
            pltpu.VMEM((_LCAP,), jnp.int32),
            pltpu.VMEM((_LCAP,), jnp.int32),
            pltpu.VMEM((_LCAP,), jnp.int32),
            pltpu.VMEM((_F, _TC), jnp.float32),
            pltpu.VMEM((_F, _TC), jnp.float32),
            pltpu.VMEM((_SCAP, _TC), jnp.float32),
            pltpu.VMEM((_SCAP, _TC), jnp.float32),
            pltpu.VMEM((_SCAP,), jnp.int32),
            pltpu.VMEM((_SCAP,), jnp.int32),
            pltpu.VMEM((_SCAP,), jnp.int32),
            pltpu.VMEM((_SCAP,), jnp.int32),
            pltpu.SemaphoreType.DMA,
            pltpu.SemaphoreType.DMA,
        ],
    )
    rows_p, rows_q = k1(pT, qT, pi, qi)

    k2 = pl.kernel(
        _k2_body,
        mesh=mesh,
        compiler_params=params,
        out_type=jax.ShapeDtypeStruct((_B,), jnp.float32),
        scratch_types=[
            pltpu.VMEM((_NCH, _CH), jnp.int32),
            pltpu.VMEM((_NCH, _CH), jnp.int32),
            pltpu.VMEM((_HB, _TC), jnp.float32),
            pltpu.VMEM((_HB, _TC), jnp.float32),
            pltpu.VMEM((_BPW,), jnp.float32),
            pltpu.VMEM((_BPW,), jnp.float32),
            pltpu.VMEM((_BPW,), jnp.float32),
            pltpu.SemaphoreType.DMA,
        ],
    )
    return k2(rows_p, rows_q, p2, q2, p_bias.reshape(-1),
              poll_bias.reshape(-1))


# speedup vs baseline: 1.3277x; 1.0298x over previous
import functools

import jax
import jax.numpy as jnp
from jax import lax
from jax.experimental import pallas as pl
from jax.experimental.pallas import tpu as pltpu
from jax.experimental.pallas import tpu_sc as plsc

_NC = 2
_NS = 16
_L = 16
_NW = _NC * _NS
_B = 16384
_F = 64
_N = 100000
_TC = 128
_NCOL = (_N + _TC - 1) // _TC
_CPW = 26
_LCAP = 720
_SCAP = 64
_BPW = _B // _NW
_CH = 128
_NCH = _BPW // _CH
_HB = _BPW // 2
_G = _HB // _L
_BD = _B + _TC
_SENT = 0x7FFF0000
_DUMMY = _B


def _k1_body(pT, qT, pidx_hbm, qidx_hbm, out_p, out_q,
             idx_v, lidx_p, lpos_p, lidx_q, lpos_q,
             slab_p, slab_q, ebuf_p, ebuf_q,
             csub_p, psub_p, csub_q, psub_q,
             slab_p2, slab_q2, ebuf_p2, ebuf_q2,
             csub_p2, psub_p2, csub_q2, psub_q2,
             sem_in, sem_out, sem_in2, sem_out2):
    c = lax.axis_index("c")
    s = lax.axis_index("s")
    wid = s * _NC + c
    lo = wid * _CPW * _TC
    hi = lo + _CPW * _TC
    iota = lax.iota(jnp.int32, _L)

    def fill(t, carry):
        off = pl.ds(t * _L, _L)
        sv = jnp.full((_L,), _SENT, jnp.int32)
        lidx_p[off] = sv
        lidx_q[off] = sv
        lpos_p[off] = sv
        lpos_q[off] = sv
        return carry
    lax.fori_loop(0, _LCAP // _L, fill, 0)

    pltpu.sync_copy(pidx_hbm, idx_v.at[0])
    pltpu.sync_copy(qidx_hbm, idx_v.at[1])

    def scan(t, carry):
        offp, offq = carry
        sl = pl.ds(t * _L, _L)
        pos = t * _L + iota
        op = jnp.minimum(offp, _LCAP - _L)
        oq = jnp.minimum(offq, _LCAP - _L)
        vp = idx_v[0, sl]
        mp = (vp >= lo) & (vp < hi)
        plsc.store_compressed(lidx_p.at[pl.ds(op, _L)], vp, mask=mp)
        plsc.store_compressed(lpos_p.at[pl.ds(op, _L)], pos, mask=mp)
        np_ = plsc.all_reduce_population_count(mp)[0]
        vq = idx_v[1, sl]
        mq = (vq >= lo) & (vq < hi)
        plsc.store_compressed(lidx_q.at[pl.ds(oq, _L)], vq, mask=mq)
        plsc.store_compressed(lpos_q.at[pl.ds(oq, _L)], pos, mask=mq)
        nq = plsc.all_reduce_population_count(mq)[0]
        return offp + np_, offq + nq

    lax.fori_loop(0, _B // _L, scan, (jnp.int32(0), jnp.int32(0)))

    def col_of(c_loc):
        return pl.multiple_of(
            jnp.minimum(wid * _CPW + c_loc, _NCOL - 1) * _TC, _TC)

    def fire_in(c_loc, slabs, sem):
        col0 = col_of(c_loc)
        pltpu.async_copy(
            pT.at[pl.ds(0, _F), pl.ds(col0, _TC)], slabs[0], sem)
        pltpu.async_copy(
            qT.at[pl.ds(0, _F), pl.ds(col0, _TC)], slabs[1], sem)

    def drain(dst, sem):
        pltpu.make_async_copy(out_p.at[pl.ds(0, dst.shape[0])], dst, sem).wait()

    def process(c_loc, bufs):
        slabs, ebufs, csubs, psubs, sem_in, sem_out = bufs
        col0 = col_of(c_loc)
        drain(slabs[0], sem_in)
        drain(slabs[1], sem_in)
        drain(ebufs[0], sem_out)
        drain(ebufs[1], sem_out)
        dums = jnp.full((_L,), _DUMMY, jnp.int32) + wid
        zeros = jnp.zeros((_L,), jnp.int32)
        for tb in range(2):
            for t in range(_SCAP // _L):
                psubs[tb][pl.ds(t * _L, _L)] = dums
                csubs[tb][pl.ds(t * _L, _L)] = zeros
        for tb, (lidx, lpos) in enumerate(
                ((lidx_p, lpos_p), (lidx_q, lpos_q))):
            def filt(t, off2):
                sl = pl.ds(t * _L, _L)
                v = lidx[sl]
                po = lpos[sl]
                m = (v >= col0) & (v < col0 + _TC)
                o2 = jnp.minimum(off2, _SCAP - _L)
                plsc.store_compressed(
                    csubs[tb].at[pl.ds(o2, _L)], v - col0, mask=m)
                plsc.store_compressed(
                    psubs[tb].at[pl.ds(o2, _L)], po, mask=m)
                return off2 + plsc.all_reduce_population_count(m)[0]
            n2 = lax.fori_loop(0, _LCAP // _L, filt, jnp.int32(0))
            for t in range(_SCAP // _L):
                @pl.when(t * _L < n2)
                def _():
                    cols = csubs[tb][pl.ds(t * _L, _L)]
                    for r in range(_L):
                        colr = jnp.full((_L,), cols[r], jnp.int32)
                        for k in range(_F // _L):
                            vals = plsc.load_gather(
                                slabs[tb], [k * _L + iota, colr])
                            ebufs[tb][t * _L + r, pl.ds(k * _L, _L)] = vals
        pltpu.async_copy(ebufs[0], out_p.at[psubs[0]], sem_out)
        pltpu.async_copy(ebufs[1], out_q.at[psubs[1]], sem_out)

    bufs_a = ((slab_p, slab_q), (ebuf_p, ebuf_q),
              (csub_p, csub_q), (psub_p, psub_q), sem_in, sem_out)
    bufs_b = ((slab_p2, slab_q2), (ebuf_p2, ebuf_q2),
              (csub_p2, csub_q2), (psub_p2, psub_q2), sem_in2, sem_out2)

    fire_in(0, bufs_a[0], sem_in)
    fire_in(1, bufs_b[0], sem_in2)
    dums0 = jnp.full((_L,), _DUMMY, jnp.int32) + wid
    for bufs in (bufs_a, bufs_b):
        for tb in range(2):
            for t in range(_SCAP // _L):
                bufs[3][tb][pl.ds(t * _L, _L)] = dums0
        pltpu.async_copy(bufs[1][0], out_p.at[bufs[3][0]], bufs[5])
        pltpu.async_copy(bufs[1][1], out_q.at[bufs[3][1]], bufs[5])

    def slab_loop(i, carry):
        process(2 * i, bufs_a)
        fire_in(2 * i + 2, bufs_a[0], sem_in)
        process(2 * i + 1, bufs_b)
        fire_in(2 * i + 3, bufs_b[0], sem_in2)
        return carry

    lax.fori_loop(0, _CPW // 2, slab_loop, 0)

    for bufs in (bufs_a, bufs_b):
        drain(bufs[0][0], bufs[4])
        drain(bufs[0][1], bufs[4])
        drain(bufs[1][0], bufs[5])
        drain(bufs[1][1], bufs[5])


def _k2_body(rows_p_hbm, rows_q_hbm, p2_hbm, q2_hbm, pb_hbm, qb_hbm, out_hbm,
             idx_p, idx_q, pe_v, qe_v, pb_v, qb_v, out_v, sem):
    c = lax.axis_index("c")
    s = lax.axis_index("s")
    wid = s * _NC + c
    base = wid * _BPW

    pltpu.sync_copy(p2_hbm.at[wid], idx_p)
    pltpu.sync_copy(q2_hbm.at[wid], idx_q)

    bias_copies = []
    for j in range(_NCH):
        dst = pl.ds(j * _CH, _CH)
        bias_copies.append(
            pltpu.async_copy(pb_hbm.at[idx_p.at[j]], pb_v.at[dst], sem))
        bias_copies.append(
            pltpu.async_copy(qb_hbm.at[idx_q.at[j]], qb_v.at[dst], sem))

    iota = lax.iota(jnp.int32, _L)

    for h in range(2):
        rbase = base + h * _HB
        cp_p = pltpu.async_copy(
            rows_p_hbm.at[pl.ds(rbase, _HB)], pe_v, sem)
        cp_q = pltpu.async_copy(
            rows_q_hbm.at[pl.ds(rbase, _HB)], qe_v, sem)
        cp_p.wait()
        cp_q.wait()
        if h == 0:
            for cp in bias_copies:
                cp.wait()

        def group(g, carry):
            dots = jnp.zeros((_L,), jnp.float32)
            for r in range(_L):
                rloc = g * _L + r
                s_acc = None
                for k in range(_F // _L):
                    a = pe_v[rloc, pl.ds(k * _L, _L)]
                    b = qe_v[rloc, pl.ds(k * _L, _L)]
                    s_acc = a * b if s_acc is None else s_acc + a * b
                dot = jnp.sum(s_acc)
                dots = jnp.where(iota == r, dot, dots)
            off = pl.ds(h * _HB + g * _L, _L)
            x = dots + pb_v[off] + qb_v[off]
            out_v[off] = 1.0 / (1.0 + jnp.exp(-x))
            return carry

        lax.fori_loop(0, _G, group, 0)

    pltpu.sync_copy(out_v, out_hbm.at[pl.ds(base, _BPW)])


@jax.jit
def kernel(p, poll, p_embed, p_bias, poll_embed, poll_bias):
    pi = p.astype(jnp.int32)
    qi = poll.astype(jnp.int32)
    p2 = pi.reshape(_NW, _NCH, _CH)
    q2 = qi.reshape(_NW, _NCH, _CH)
    pT = p_embed.T
    qT = poll_embed.T
    mesh = plsc.VectorSubcoreMesh(core_axis_name="c", subcore_axis_name="s")
    params = pltpu.CompilerParams(
        needs_layout_passes=False, use_tc_tiling_on_sc=True)

    k1 = pl.kernel(
        _k1_body,
        mesh=mesh,
        compiler_params=params,
        out_type=(
            jax.ShapeDtypeStruct((_BD, _TC), jnp.float32),
            jax.ShapeDtypeStruct((_BD, _TC), jnp.float32),
        ),
        scratch_types=[
            pltpu.VMEM((2, _B), jnp.int32),
            pltpu.VMEM((_LCAP,), jnp.int32),
            pltpu.VMEM((_LCAP,), jnp.int32),
            pltpu.VMEM((_LCAP,), jnp.int32),
            pltpu.VMEM((_LCAP,), jnp.int32),
            pltpu.VMEM((_F, _TC), jnp.float32),
            pltpu.VMEM((_F, _TC), jnp.float32),
            pltpu.VMEM((_SCAP, _TC), jnp.float32),
            pltpu.VMEM((_SCAP, _TC), jnp.float32),
            pltpu.VMEM((_SCAP,), jnp.int32),
            pltpu.VMEM((_SCAP,), jnp.int32),
            pltpu.VMEM((_SCAP,), jnp.int32),
            pltpu.VMEM((_SCAP,), jnp.int32),
            pltpu.VMEM((_F, _TC), jnp.float32),
            pltpu.VMEM((_F, _TC), jnp.float32),
            pltpu.VMEM((_SCAP, _TC), jnp.float32),
            pltpu.VMEM((_SCAP, _TC), jnp.float32),
            pltpu.VMEM((_SCAP,), jnp.int32),
            pltpu.VMEM((_SCAP,), jnp.int32),
            pltpu.VMEM((_SCAP,), jnp.int32),
            pltpu.VMEM((_SCAP,), jnp.int32),
            pltpu.SemaphoreType.DMA,
            pltpu.SemaphoreType.DMA,
            pltpu.SemaphoreType.DMA,
            pltpu.SemaphoreType.DMA,
        ],
    )
    rows_p, rows_q = k1(pT, qT, pi, qi)

    k2 = pl.kernel(
        _k2_body,
        mesh=mesh,
        compiler_params=params,
        out_type=jax.ShapeDtypeStruct((_B,), jnp.float32),
        scratch_types=[
            pltpu.VMEM((_NCH, _CH), jnp.int32),
            pltpu.VMEM((_NCH, _CH), jnp.int32),
            pltpu.VMEM((_HB, _TC), jnp.float32),
            pltpu.VMEM((_HB, _TC), jnp.float32),
            pltpu.VMEM((_BPW,), jnp.float32),
            pltpu.VMEM((_BPW,), jnp.float32),
            pltpu.VMEM((_BPW,), jnp.float32),
            pltpu.SemaphoreType.DMA,
        ],
    )
    return k2(rows_p, rows_q, p2, q2, p_bias.reshape(-1),
              poll_bias.reshape(-1))


# speedup vs baseline: 2.2501x; 1.6946x over previous
import functools

import jax
import jax.numpy as jnp
from jax import lax
from jax.experimental import pallas as pl
from jax.experimental.pallas import tpu as pltpu
from jax.experimental.pallas import tpu_sc as plsc

_NC = 2
_NS = 16
_L = 16
_NW = _NC * _NS
_B = 16384
_F = 64
_BPW = _B // _NW
_CH = 128
_NCH = _BPW // _CH
_G = _BPW // _L


def _body(p_ref, poll_ref, pe_hbm, pb_hbm, qe_hbm, qb_hbm, out_hbm,
          idx_p, idx_q, pe_v, qe_v, pb_v, qb_v, out_v, sem):
    c = lax.axis_index("c")
    s = lax.axis_index("s")
    wid = s * _NC + c
    row0 = wid * _NCH
    base = wid * _BPW

    pltpu.sync_copy(p_ref.at[pl.ds(row0, _NCH)], idx_p)
    pltpu.sync_copy(poll_ref.at[pl.ds(row0, _NCH)], idx_q)

    copies = []
    for j in range(_NCH):
        dst = pl.ds(j * _CH, _CH)
        copies.append(pltpu.async_copy(pe_hbm.at[idx_p.at[j]], pe_v.at[dst], sem))
        copies.append(pltpu.async_copy(qe_hbm.at[idx_q.at[j]], qe_v.at[dst], sem))
        copies.append(pltpu.async_copy(pb_hbm.at[idx_p.at[j]], pb_v.at[dst], sem))
        copies.append(pltpu.async_copy(qb_hbm.at[idx_q.at[j]], qb_v.at[dst], sem))
    for cp in copies:
        cp.wait()

    iota = lax.iota(jnp.int32, _L)

    def group(g, carry):
        dots = jnp.zeros((_L,), jnp.float32)
        for r in range(_L):
            row = g * _L + r
            s_acc = None
            for k in range(_F // _L):
                a = pe_v[row, pl.ds(k * _L, _L)]
                b = qe_v[row, pl.ds(k * _L, _L)]
                s_acc = a * b if s_acc is None else s_acc + a * b
            dot = jnp.sum(s_acc)
            dots = jnp.where(iota == r, dot, dots)
        off = pl.ds(g * _L, _L)
        x = dots + pb_v[off] + qb_v[off]
        out_v[off] = 1.0 / (1.0 + jnp.exp(-x))
        return carry

    lax.fori_loop(0, _G, group, 0)

    pltpu.sync_copy(out_v, out_hbm.at[pl.ds(base, _BPW)])


@jax.jit
def kernel(p, poll, p_embed, p_bias, poll_embed, poll_bias):
    p2 = p.astype(jnp.int32).reshape(_NW * _NCH, _CH)
    q2 = poll.astype(jnp.int32).reshape(_NW * _NCH, _CH)
    mesh = plsc.VectorSubcoreMesh(core_axis_name="c", subcore_axis_name="s")
    run = pl.kernel(
        _body,
        mesh=mesh,
        compiler_params=pltpu.CompilerParams(
            needs_layout_passes=False, use_tc_tiling_on_sc=False),
        out_type=jax.ShapeDtypeStruct((_B,), jnp.float32),
        scratch_types=[
            pltpu.VMEM((_NCH, _CH), jnp.int32),
            pltpu.VMEM((_NCH, _CH), jnp.int32),
            pltpu.VMEM((_BPW, _F), jnp.float32),
            pltpu.VMEM((_BPW, _F), jnp.float32),
            pltpu.VMEM((_BPW,), jnp.float32),
            pltpu.VMEM((_BPW,), jnp.float32),
            pltpu.VMEM((_BPW,), jnp.float32),
            pltpu.SemaphoreType.DMA,
        ],
    )
    return run(p2, q2, p_embed, p_bias.reshape(-1), poll_embed,
               poll_bias.reshape(-1))
